# baseline (device time: 106788 ns/iter reference)
import jax
import jax.numpy as jnp
from jax import lax
from jax.experimental import pallas as pl
from jax.experimental.pallas import tpu as pltpu

N_DEV = 16
NZ = 4
NP = 4
M_HALF = 128

N_C = 7


def kernel(x, w_mat, scale_x, scale_w):
    m_per, k = x.shape
    k2, n_per = w_mat.shape
    assert k == k2 and m_per == 2 * M_HALF

    def body(x_ref, w_ref, sx_ref, sw_ref, out_ref,
             own_ref, upb_ref, dnb_ref, ff_ref, bf_ref, fh_ref, bh_ref, w8_ref,
             up_s, up_r, dn_s, dn_r,
             ffs, ffr, bfs, bfr, fhs, fhr, bhs, bhr):
        my = lax.axis_index("i")
        z = my // NP
        j = lax.rem(my, NP)
        base = my - j
        right_p = base + lax.rem(j + 1, NP)
        left_p = base + lax.rem(j + 3, NP)
        up_id = lax.rem(my + NP, N_DEV)
        dn_id = lax.rem(my + N_DEV - NP, N_DEV)
        has_up = z < NZ - 1
        has_dn = z > 0

        barrier_sem = pltpu.get_barrier_semaphore()
        for nbr in (left_p, right_p, up_id, dn_id):
            pl.semaphore_signal(
                barrier_sem, inc=1,
                device_id=(nbr,), device_id_type=pl.DeviceIdType.MESH,
            )
        pl.semaphore_wait(barrier_sem, 4)

        scale = sx_ref[0] * sw_ref[0]
        own_ref[...] = x_ref[...].astype(jnp.float8_e4m3fn)

        def col_msg(s, buf_ref, ssems, rsems, dev):
            src = own_ref.at[:, :] if s == 0 else \
                buf_ref.at[pl.ds((s - 1) * m_per, m_per), :]
            return pltpu.make_async_remote_copy(
                src_ref=src,
                dst_ref=buf_ref.at[pl.ds(s * m_per, m_per), :],
                send_sem=ssems.at[s],
                recv_sem=rsems.at[s],
                device_id=(dev,),
                device_id_type=pl.DeviceIdType.MESH,
            )

        up_rdmas = [col_msg(s, upb_ref, up_s, up_r, up_id) for s in range(NZ - 1)]
        dn_rdmas = [col_msg(s, dnb_ref, dn_s, dn_r, dn_id) for s in range(NZ - 1)]

        def o_of(c):
            if c == 0:
                return z
            if c % 2 == 1:
                return z - 1 - (c - 1) // 2
            return z + 1 + (c - 2) // 2

        def valid(c):
            if c == 0:
                return None
            if c % 2 == 1:
                return z > (c - 1) // 2
            return z < NZ - 1 - (c - 2) // 2

        def src_full(c):
            if c == 0:
                return own_ref.at[:, :]
            t = (c - 1) // 2 if c % 2 == 1 else (c - 2) // 2
            buf = upb_ref if c % 2 == 1 else dnb_ref
            return buf.at[pl.ds(t * m_per, m_per), :]

        ff_rdmas, bf_rdmas, fh_rdmas, bh_rdmas = [], [], [], []
        for c in range(N_C):
            o = o_of(c)
            ff_rdmas.append(pltpu.make_async_remote_copy(
                src_ref=src_full(c),
                dst_ref=ff_ref.at[pl.ds(o * m_per, m_per), :],
                send_sem=ffs.at[c], recv_sem=ffr.at[c],
                device_id=(right_p,), device_id_type=pl.DeviceIdType.MESH))
            bf_rdmas.append(pltpu.make_async_remote_copy(
                src_ref=src_full(c),
                dst_ref=bf_ref.at[pl.ds(o * m_per, m_per), :],
                send_sem=bfs.at[c], recv_sem=bfr.at[c],
                device_id=(left_p,), device_id_type=pl.DeviceIdType.MESH))
            fh_rdmas.append(pltpu.make_async_remote_copy(
                src_ref=ff_ref.at[pl.ds(o * m_per, M_HALF), :],
                dst_ref=fh_ref.at[pl.ds(o * M_HALF, M_HALF), :],
                send_sem=fhs.at[c], recv_sem=fhr.at[c],
                device_id=(right_p,), device_id_type=pl.DeviceIdType.MESH))
            bh_rdmas.append(pltpu.make_async_remote_copy(
                src_ref=bf_ref.at[pl.ds(o * m_per + M_HALF, M_HALF), :],
                dst_ref=bh_ref.at[pl.ds(o * M_HALF, M_HALF), :],
                send_sem=bhs.at[c], recv_sem=bhr.at[c],
                device_id=(left_p,), device_id_type=pl.DeviceIdType.MESH))

        def gemm_full(chunk, p):
            acc = jnp.dot(chunk, w8_ref[...], preferred_element_type=jnp.float32)
            out_ref[pl.ds(p * m_per, m_per), :] = acc * scale

        def gemm_half(half, p, is_b):
            acc = jnp.dot(half, w8_ref[...], preferred_element_type=jnp.float32)
            off = p * m_per + (M_HALF if is_b else 0)
            out_ref[pl.ds(off, M_HALF), :] = acc * scale

        @pl.when(has_up)
        def _():
            up_rdmas[0].start()

        @pl.when(has_dn)
        def _():
            dn_rdmas[0].start()

        ff_rdmas[0].start()
        bf_rdmas[0].start()

        w8_ref[...] = w_ref[...].astype(jnp.float8_e4m3fn)
        gemm_full(own_ref[...], my)

        for c in range(N_C):
            if c >= 1:
                t = (c - 1) // 2 if c % 2 == 1 else (c - 2) // 2
                is_up = c % 2 == 1

                @pl.when(valid(c))
                def _(c=c, t=t, is_up=is_up):
                    if is_up:
                        up_rdmas[t].wait_recv()
                        if t + 1 < NZ - 1:
                            @pl.when(has_up)
                            def _():
                                up_rdmas[t + 1].start()
                        chunk = upb_ref[pl.ds(t * m_per, m_per), :]
                        gemm_full(chunk, my - NP * (1 + t))
                    else:
                        dn_rdmas[t].wait_recv()
                        if t + 1 < NZ - 1:
                            @pl.when(has_dn)
                            def _():
                                dn_rdmas[t + 1].start()
                        chunk = dnb_ref[pl.ds(t * m_per, m_per), :]
                        gemm_full(chunk, my + NP * (1 + t))
                    ff_rdmas[c].start()
                    bf_rdmas[c].start()

            def full_recvs(c=c):
                o = o_of(c)
                ff_rdmas[c].wait_recv()
                fh_rdmas[c].start()
                bf_rdmas[c].wait_recv()
                bh_rdmas[c].start()
                gemm_full(ff_ref[pl.ds(o * m_per, m_per), :],
                          NP * o + lax.rem(j + 3, NP))
                gemm_full(bf_ref[pl.ds(o * m_per, m_per), :],
                          NP * o + lax.rem(j + 1, NP))

            if c == 0:
                full_recvs()
            else:
                pl.when(valid(c))(full_recvs)

            if c >= 1:
                def half_recvs(c=c - 1):
                    o = o_of(c)
                    p2 = NP * o + lax.rem(j + 2, NP)
                    fh_rdmas[c].wait_recv()
                    gemm_half(fh_ref[pl.ds(o * M_HALF, M_HALF), :], p2, False)
                    bh_rdmas[c].wait_recv()
                    gemm_half(bh_ref[pl.ds(o * M_HALF, M_HALF), :], p2, True)

                if c == 1:
                    half_recvs()
                else:
                    pl.when(valid(c - 1))(half_recvs)

        @pl.when(valid(N_C - 1))
        def _():
            c = N_C - 1
            o = o_of(c)
            p2 = NP * o + lax.rem(j + 2, NP)
            fh_rdmas[c].wait_recv()
            gemm_half(fh_ref[pl.ds(o * M_HALF, M_HALF), :], p2, False)
            bh_rdmas[c].wait_recv()
            gemm_half(bh_ref[pl.ds(o * M_HALF, M_HALF), :], p2, True)

        @pl.when(has_up)
        def _():
            up_rdmas[0].wait_send()

        @pl.when(has_dn)
        def _():
            dn_rdmas[0].wait_send()

        for t in range(NZ - 2):
            @pl.when(jnp.logical_and(z > t, has_up))
            def _(t=t):
                up_rdmas[t + 1].wait_send()

            @pl.when(jnp.logical_and(z < NZ - 1 - t, has_dn))
            def _(t=t):
                dn_rdmas[t + 1].wait_send()

        for c in range(N_C):
            def drain(c=c):
                ff_rdmas[c].wait_send()
                bf_rdmas[c].wait_send()
                fh_rdmas[c].wait_send()
                bh_rdmas[c].wait_send()

            if c == 0:
                drain()
            else:
                pl.when(valid(c))(drain)

    return pl.pallas_call(
        body,
        out_shape=jax.ShapeDtypeStruct((N_DEV * m_per, n_per), jnp.float32),
        in_specs=[
            pl.BlockSpec(memory_space=pltpu.VMEM),
            pl.BlockSpec(memory_space=pltpu.VMEM),
            pl.BlockSpec(memory_space=pltpu.SMEM),
            pl.BlockSpec(memory_space=pltpu.SMEM),
        ],
        out_specs=pl.BlockSpec(memory_space=pltpu.VMEM),
        scratch_shapes=[
            pltpu.VMEM((m_per, k), jnp.float8_e4m3fn),
            pltpu.VMEM(((NZ - 1) * m_per, k), jnp.float8_e4m3fn),
            pltpu.VMEM(((NZ - 1) * m_per, k), jnp.float8_e4m3fn),
            pltpu.VMEM((NZ * m_per, k), jnp.float8_e4m3fn),
            pltpu.VMEM((NZ * m_per, k), jnp.float8_e4m3fn),
            pltpu.VMEM((NZ * M_HALF, k), jnp.float8_e4m3fn),
            pltpu.VMEM((NZ * M_HALF, k), jnp.float8_e4m3fn),
            pltpu.VMEM((k, n_per), jnp.float8_e4m3fn),
            pltpu.SemaphoreType.DMA((NZ - 1,)),
            pltpu.SemaphoreType.DMA((NZ - 1,)),
            pltpu.SemaphoreType.DMA((NZ - 1,)),
            pltpu.SemaphoreType.DMA((NZ - 1,)),
            pltpu.SemaphoreType.DMA((N_C,)),
            pltpu.SemaphoreType.DMA((N_C,)),
            pltpu.SemaphoreType.DMA((N_C,)),
            pltpu.SemaphoreType.DMA((N_C,)),
            pltpu.SemaphoreType.DMA((N_C,)),
            pltpu.SemaphoreType.DMA((N_C,)),
            pltpu.SemaphoreType.DMA((N_C,)),
            pltpu.SemaphoreType.DMA((N_C,)),
        ],
        compiler_params=pltpu.CompilerParams(collective_id=0),
    )(x, w_mat, scale_x, scale_w)


# device time: 102463 ns/iter; 1.0422x vs baseline; 1.0422x over previous
import jax
import jax.numpy as jnp
from jax import lax
from jax.experimental import pallas as pl
from jax.experimental.pallas import tpu as pltpu

N_DEV = 16
N_MSG = 15
M_HALF = 128


def kernel(x, w_mat, scale_x, scale_w):
    m_per, k = x.shape
    k2, n_per = w_mat.shape
    assert k == k2 and m_per == 2 * M_HALF

    def body(x_ref, w_ref, sx_ref, sw_ref, out_ref,
             fwd_ref, bwd_ref, w8_ref,
             fs_sems, fr_sems, bs_sems, br_sems):
        my = lax.axis_index("i")
        left = lax.rem(my + N_DEV - 1, N_DEV)
        right = lax.rem(my + 1, N_DEV)

        barrier_sem = pltpu.get_barrier_semaphore()
        for nbr in (left, right):
            pl.semaphore_signal(
                barrier_sem, inc=1,
                device_id=(nbr,), device_id_type=pl.DeviceIdType.MESH,
            )
        pl.semaphore_wait(barrier_sem, 2)

        scale = sx_ref[0] * sw_ref[0]

        x8 = x_ref[...].astype(jnp.float8_e4m3fn)

        def mk(buf_ref, s, ssems, rsems, dev):
            return pltpu.make_async_remote_copy(
                src_ref=buf_ref.at[pl.ds(s * M_HALF, M_HALF), :],
                dst_ref=buf_ref.at[pl.ds((s + 2) * M_HALF, M_HALF), :],
                send_sem=ssems.at[s],
                recv_sem=rsems.at[s],
                device_id=(dev,),
                device_id_type=pl.DeviceIdType.MESH,
            )

        fwd_rdmas = [mk(fwd_ref, s, fs_sems, fr_sems, right) for s in range(N_MSG)]
        bwd_rdmas = [mk(bwd_ref, s, bs_sems, br_sems, left) for s in range(N_MSG)]

        fwd_ref[pl.ds(0, M_HALF), :] = x8[:M_HALF]
        fwd_rdmas[0].start()
        bwd_ref[pl.ds(0, M_HALF), :] = x8[M_HALF:]
        bwd_rdmas[0].start()
        fwd_ref[pl.ds(M_HALF, M_HALF), :] = x8[M_HALF:]
        fwd_rdmas[1].start()
        bwd_ref[pl.ds(M_HALF, M_HALF), :] = x8[:M_HALF]
        bwd_rdmas[1].start()

        w8_ref[...] = w_ref[...].astype(jnp.float8_e4m3fn)
        acc = jnp.dot(x8, w8_ref[...], preferred_element_type=jnp.float32)
        out_ref[pl.ds(my * m_per, m_per), :] = acc * scale

        origin8 = lax.rem(my + N_DEV // 2, N_DEV)

        for s in range(N_MSG):
            fwd_rdmas[s].wait_recv()
            if s + 2 < N_MSG:
                fwd_rdmas[s + 2].start()
            if s == N_MSG - 1:
                acc = jnp.dot(fwd_ref[pl.ds(16 * M_HALF, M_HALF), :],
                              w8_ref[...], preferred_element_type=jnp.float32)
                out_ref[pl.ds(origin8 * m_per, M_HALF), :] = acc * scale
            bwd_rdmas[s].wait_recv()
            if s + 2 < N_MSG:
                bwd_rdmas[s + 2].start()

            if s % 2 == 1:
                j = (s - 1) // 2
                origin_f = lax.rem(my + N_DEV - (j + 1), N_DEV)
                chunk_f = fwd_ref[pl.ds((s + 1) * M_HALF, m_per), :]
                acc = jnp.dot(chunk_f, w8_ref[...],
                              preferred_element_type=jnp.float32)
                out_ref[pl.ds(origin_f * m_per, m_per), :] = acc * scale

                origin_b = lax.rem(my + j + 1, N_DEV)
                chunk_b = bwd_ref[pl.ds((s + 1) * M_HALF, m_per), :]
                acc = jnp.dot(chunk_b, w8_ref[...],
                              preferred_element_type=jnp.float32)
                out_ref[pl.ds(origin_b * m_per + M_HALF, M_HALF), :] = \
                    acc[:M_HALF] * scale
                out_ref[pl.ds(origin_b * m_per, M_HALF), :] = \
                    acc[M_HALF:] * scale

        acc = jnp.dot(bwd_ref[pl.ds(16 * M_HALF, M_HALF), :], w8_ref[...],
                      preferred_element_type=jnp.float32)
        out_ref[pl.ds(origin8 * m_per + M_HALF, M_HALF), :] = acc * scale

        for r in fwd_rdmas:
            r.wait_send()
        for r in bwd_rdmas:
            r.wait_send()

    n_slots = N_MSG + 2
    return pl.pallas_call(
        body,
        out_shape=jax.ShapeDtypeStruct((N_DEV * m_per, n_per), jnp.float32),
        in_specs=[
            pl.BlockSpec(memory_space=pltpu.VMEM),
            pl.BlockSpec(memory_space=pltpu.VMEM),
            pl.BlockSpec(memory_space=pltpu.SMEM),
            pl.BlockSpec(memory_space=pltpu.SMEM),
        ],
        out_specs=pl.BlockSpec(memory_space=pltpu.VMEM),
        scratch_shapes=[
            pltpu.VMEM((n_slots * M_HALF, k), jnp.float8_e4m3fn),
            pltpu.VMEM((n_slots * M_HALF, k), jnp.float8_e4m3fn),
            pltpu.VMEM((k, n_per), jnp.float8_e4m3fn),
            pltpu.SemaphoreType.DMA((N_MSG,)),
            pltpu.SemaphoreType.DMA((N_MSG,)),
            pltpu.SemaphoreType.DMA((N_MSG,)),
            pltpu.SemaphoreType.DMA((N_MSG,)),
        ],
        compiler_params=pltpu.CompilerParams(collective_id=0),
    )(x, w_mat, scale_x, scale_w)
